# Initial kernel scaffold; baseline (speedup 1.0000x reference)
import jax
import jax.numpy as jnp
from jax.experimental import pallas as pl


def kernel(points, W_d, b_d, W_a, b_a):
    raise NotImplementedError("write your pallas kernel here")


# speedup vs baseline: 1.3042x; 1.3042x over previous
import numpy as np
import jax
import jax.numpy as jnp
from jax.experimental import pallas as pl
from jax.experimental.pallas import tpu as pltpu

_HID = 256
_HALF = _HID // 2
_SIGMA_D = 0.2
_FACTOR_A = 180.0 / (15.0 * np.pi)
_K = 3
_N = 256
_R = 8


def _emb_linear(x, w_ref, div_ref):
    om = x.reshape(_R, _N, 1) * div_ref[:][None, :, :]
    s = jnp.sin(om).reshape(_R * _N, _HALF)
    c = jnp.cos(om).reshape(_R * _N, _HALF)
    return (jnp.dot(s, w_ref[0:_HALF, :], preferred_element_type=jnp.float32)
            + jnp.dot(c, w_ref[_HALF:, :], preferred_element_type=jnp.float32))


def _block_kernel(pts_ref, ptT_ref, wd_ref, wa_ref, bias_ref, div_ref, out_ref):
    i = pl.program_id(0)
    ptT = ptT_ref[:]
    prow = pts_ref[pl.ds(i * _R, _R), :]

    xy = jnp.dot(prow, ptT, preferred_element_type=jnp.float32)
    x2 = jnp.sum(prow * prow, axis=1, keepdims=True)
    y2 = jnp.sum(ptT * ptT, axis=0, keepdims=True)
    dist = jnp.sqrt(jnp.maximum(x2 - 2.0 * xy + y2, 0.0))

    iota = jax.lax.broadcasted_iota(jnp.int32, (_R, _N), 1)
    px_all = ptT[0:1, :]
    py_all = ptT[1:2, :]
    pz_all = ptT[2:3, :]
    px_row = prow[:, 0:1]
    py_row = prow[:, 1:2]
    pz_row = prow[:, 2:3]
    ax = px_all - px_row
    ay = py_all - py_row
    az = pz_all - pz_row

    d = dist
    refvecs = []
    for t in range(_K + 1):
        vmin = jnp.min(d, axis=1, keepdims=True)
        idx = jnp.min(jnp.where(d == vmin, iota, _N), axis=1, keepdims=True)
        sel = iota == idx
        d = jnp.where(sel, jnp.float32(np.inf), d)
        if t > 0:
            rx = jnp.sum(jnp.where(sel, px_all, 0.0), axis=1, keepdims=True) - px_row
            ry = jnp.sum(jnp.where(sel, py_all, 0.0), axis=1, keepdims=True) - py_row
            rz = jnp.sum(jnp.where(sel, pz_all, 0.0), axis=1, keepdims=True) - pz_row
            refvecs.append((rx, ry, rz))

    out = _emb_linear(dist * (1.0 / _SIGMA_D), wd_ref, div_ref)

    m = None
    for rx, ry, rz in refvecs:
        cx = ry * az - rz * ay
        cy = rz * ax - rx * az
        cz = rx * ay - ry * ax
        sinv = jnp.sqrt(cx * cx + cy * cy + cz * cz)
        cosv = rx * ax + ry * ay + rz * az
        ang = jnp.arctan2(sinv, cosv) * _FACTOR_A
        e = _emb_linear(ang, wa_ref, div_ref)
        m = e if m is None else jnp.maximum(m, e)

    out = out + m + bias_ref[:]
    out_ref[0] = out.reshape(_R, _N, _HID)


def kernel(points, W_d, b_d, W_a, b_a):
    B, N, _ = points.shape
    assert B == 1 and N == _N
    pts = points[0]
    ptT = pts.T
    wd_p = jnp.concatenate([W_d[:, 0::2], W_d[:, 1::2]], axis=1).T
    wa_p = jnp.concatenate([W_a[:, 0::2], W_a[:, 1::2]], axis=1).T
    bias = (b_d + b_a)[None, :]
    div = np.exp(np.arange(0, _HID, 2, dtype=np.float32)
                 * (-np.log(10000.0) / _HID)).astype(np.float32)[None, :]
    div = jnp.asarray(div)

    out = pl.pallas_call(
        _block_kernel,
        grid=(N // _R,),
        in_specs=[
            pl.BlockSpec((_N, 3), lambda i: (0, 0)),
            pl.BlockSpec((3, _N), lambda i: (0, 0)),
            pl.BlockSpec((_HID, _HID), lambda i: (0, 0)),
            pl.BlockSpec((_HID, _HID), lambda i: (0, 0)),
            pl.BlockSpec((1, _HID), lambda i: (0, 0)),
            pl.BlockSpec((1, _HALF), lambda i: (0, 0)),
        ],
        out_specs=pl.BlockSpec((1, _R, _N, _HID), lambda i: (0, i, 0, 0)),
        out_shape=jax.ShapeDtypeStruct((B, N, N, _HID), jnp.float32),
    )(pts, ptT, wd_p, wa_p, bias, div)
    return out


# speedup vs baseline: 3.2333x; 2.4791x over previous
import numpy as np
import jax
import jax.numpy as jnp
from jax.experimental import pallas as pl
from jax.experimental.pallas import tpu as pltpu

_HID = 256
_HALF = _HID // 2
_SIGMA_D = 0.2
_FACTOR_A = 180.0 / (15.0 * np.pi)
_K = 3
_N = 256
_R = 8


_INV2PI = float(1.0 / (2.0 * np.pi))
_P2HI = float(np.float32(2.0 * np.pi))
_P2LO = float(2.0 * np.pi - np.float64(np.float32(2.0 * np.pi)))
_SINC = (9.9999999447e-01, -1.6666664578e-01, 8.3333103669e-03, -1.9840154698e-04,
         2.7529445801e-06, -2.4676915712e-08, 1.3451342817e-10)
_COSC = (9.9999998919e-01, -4.9999989197e-01, 4.1666490344e-02, -1.3887808531e-03,
         2.4769981168e-05, -2.7079919842e-07, 1.7248124595e-09)


def _sincos(om):
    k = jnp.round(om * _INV2PI)
    r = om - k * _P2HI
    r = r - k * _P2LO
    t = r * r
    s = _SINC[6]
    c = _COSC[6]
    for j in range(5, -1, -1):
        s = s * t + _SINC[j]
        c = c * t + _COSC[j]
    return r * s, c


def _emb_linear(x, w_ref, div_ref):
    om = x.reshape(_R, _N, 1) * div_ref[:][None, :, :]
    s, c = _sincos(om)
    s = s.reshape(_R * _N, _HALF)
    c = c.reshape(_R * _N, _HALF)
    return (jnp.dot(s, w_ref[0:_HALF, :], preferred_element_type=jnp.float32)
            + jnp.dot(c, w_ref[_HALF:, :], preferred_element_type=jnp.float32))


def _block_kernel(pts_ref, ptT_ref, wd_ref, wa_ref, bias_ref, div_ref, out_ref):
    i = pl.program_id(0)
    ptT = ptT_ref[:]
    prow = pts_ref[pl.ds(i * _R, _R), :]

    xy = jnp.dot(prow, ptT, preferred_element_type=jnp.float32)
    x2 = jnp.sum(prow * prow, axis=1, keepdims=True)
    y2 = jnp.sum(ptT * ptT, axis=0, keepdims=True)
    dist = jnp.sqrt(jnp.maximum(x2 - 2.0 * xy + y2, 0.0))

    iota = jax.lax.broadcasted_iota(jnp.int32, (_R, _N), 1)
    px_all = ptT[0:1, :]
    py_all = ptT[1:2, :]
    pz_all = ptT[2:3, :]
    px_row = prow[:, 0:1]
    py_row = prow[:, 1:2]
    pz_row = prow[:, 2:3]
    ax = px_all - px_row
    ay = py_all - py_row
    az = pz_all - pz_row

    d = dist
    refvecs = []
    for t in range(_K + 1):
        vmin = jnp.min(d, axis=1, keepdims=True)
        idx = jnp.min(jnp.where(d == vmin, iota, _N), axis=1, keepdims=True)
        sel = iota == idx
        d = jnp.where(sel, jnp.float32(np.inf), d)
        if t > 0:
            rx = jnp.sum(jnp.where(sel, px_all, 0.0), axis=1, keepdims=True) - px_row
            ry = jnp.sum(jnp.where(sel, py_all, 0.0), axis=1, keepdims=True) - py_row
            rz = jnp.sum(jnp.where(sel, pz_all, 0.0), axis=1, keepdims=True) - pz_row
            refvecs.append((rx, ry, rz))

    out = _emb_linear(dist * (1.0 / _SIGMA_D), wd_ref, div_ref)

    m = None
    for rx, ry, rz in refvecs:
        cx = ry * az - rz * ay
        cy = rz * ax - rx * az
        cz = rx * ay - ry * ax
        sinv = jnp.sqrt(cx * cx + cy * cy + cz * cz)
        cosv = rx * ax + ry * ay + rz * az
        ang = jnp.arctan2(sinv, cosv) * _FACTOR_A
        e = _emb_linear(ang, wa_ref, div_ref)
        m = e if m is None else jnp.maximum(m, e)

    out = out + m + bias_ref[:]
    out_ref[0] = out.reshape(_R, _N, _HID)


def kernel(points, W_d, b_d, W_a, b_a):
    B, N, _ = points.shape
    assert B == 1 and N == _N
    pts = points[0]
    ptT = pts.T
    wd_p = jnp.concatenate([W_d[:, 0::2], W_d[:, 1::2]], axis=1).T
    wa_p = jnp.concatenate([W_a[:, 0::2], W_a[:, 1::2]], axis=1).T
    bias = (b_d + b_a)[None, :]
    div = np.exp(np.arange(0, _HID, 2, dtype=np.float32)
                 * (-np.log(10000.0) / _HID)).astype(np.float32)[None, :]
    div = jnp.asarray(div)

    out = pl.pallas_call(
        _block_kernel,
        grid=(N // _R,),
        in_specs=[
            pl.BlockSpec((_N, 3), lambda i: (0, 0)),
            pl.BlockSpec((3, _N), lambda i: (0, 0)),
            pl.BlockSpec((_HID, _HID), lambda i: (0, 0)),
            pl.BlockSpec((_HID, _HID), lambda i: (0, 0)),
            pl.BlockSpec((1, _HID), lambda i: (0, 0)),
            pl.BlockSpec((1, _HALF), lambda i: (0, 0)),
        ],
        out_specs=pl.BlockSpec((1, _R, _N, _HID), lambda i: (0, i, 0, 0)),
        out_shape=jax.ShapeDtypeStruct((B, N, N, _HID), jnp.float32),
    )(pts, ptT, wd_p, wa_p, bias, div)
    return out


# speedup vs baseline: 3.9508x; 1.2219x over previous
import numpy as np
import jax
import jax.numpy as jnp
from jax.experimental import pallas as pl
from jax.experimental.pallas import tpu as pltpu

_HID = 256
_HALF = _HID // 2
_SIGMA_D = 0.2
_FACTOR_A = 180.0 / (15.0 * np.pi)
_K = 3
_N = 256
_R = 8


_INV2PI = float(1.0 / (2.0 * np.pi))
_P2HI = float(np.float32(2.0 * np.pi))
_SINC = (9.999793369e-01, -1.666243428e-01, 8.308974449e-03, -1.926489789e-04,
         2.147843394e-06)
_COSC = (9.999598318e-01, -4.997932701e-01, 4.149610442e-02, -1.339281429e-03,
         1.879272552e-05)


def _sincos(om):
    k = jnp.round(om * _INV2PI)
    r = om - k * _P2HI
    t = r * r
    s = _SINC[4]
    c = _COSC[4]
    for j in range(3, -1, -1):
        s = s * t + _SINC[j]
        c = c * t + _COSC[j]
    return r * s, c


def _emb_linear(x, w_ref, div_ref):
    om = x.reshape(_R, _N, 1) * div_ref[:][None, :, :]
    s, c = _sincos(om)
    s = s.reshape(_R * _N, _HALF)
    c = c.reshape(_R * _N, _HALF)
    return (jnp.dot(s, w_ref[0:_HALF, :], preferred_element_type=jnp.float32)
            + jnp.dot(c, w_ref[_HALF:, :], preferred_element_type=jnp.float32))


def _block_kernel(pts_ref, ptT_ref, wd_ref, wa_ref, bias_ref, div_ref, out_ref):
    i = pl.program_id(0)
    ptT = ptT_ref[:]
    prow = pts_ref[pl.ds(i * _R, _R), :]

    xy = jnp.dot(prow, ptT, preferred_element_type=jnp.float32)
    x2 = jnp.sum(prow * prow, axis=1, keepdims=True)
    y2 = jnp.sum(ptT * ptT, axis=0, keepdims=True)
    dist = jnp.sqrt(jnp.maximum(x2 - 2.0 * xy + y2, 0.0))

    iota = jax.lax.broadcasted_iota(jnp.int32, (_R, _N), 1)
    px_all = ptT[0:1, :]
    py_all = ptT[1:2, :]
    pz_all = ptT[2:3, :]
    px_row = prow[:, 0:1]
    py_row = prow[:, 1:2]
    pz_row = prow[:, 2:3]
    ax = px_all - px_row
    ay = py_all - py_row
    az = pz_all - pz_row

    d = dist
    refvecs = []
    for t in range(_K + 1):
        vmin = jnp.min(d, axis=1, keepdims=True)
        idx = jnp.min(jnp.where(d == vmin, iota, _N), axis=1, keepdims=True)
        sel = iota == idx
        d = jnp.where(sel, jnp.float32(np.inf), d)
        if t > 0:
            rx = jnp.sum(jnp.where(sel, px_all, 0.0), axis=1, keepdims=True) - px_row
            ry = jnp.sum(jnp.where(sel, py_all, 0.0), axis=1, keepdims=True) - py_row
            rz = jnp.sum(jnp.where(sel, pz_all, 0.0), axis=1, keepdims=True) - pz_row
            refvecs.append((rx, ry, rz))

    out = _emb_linear(dist * (1.0 / _SIGMA_D), wd_ref, div_ref)

    m = None
    for rx, ry, rz in refvecs:
        cx = ry * az - rz * ay
        cy = rz * ax - rx * az
        cz = rx * ay - ry * ax
        sinv = jnp.sqrt(cx * cx + cy * cy + cz * cz)
        cosv = rx * ax + ry * ay + rz * az
        ang = jnp.arctan2(sinv, cosv) * _FACTOR_A
        e = _emb_linear(ang, wa_ref, div_ref)
        m = e if m is None else jnp.maximum(m, e)

    out = out + m + bias_ref[:]
    out_ref[0] = out.reshape(_R, _N, _HID)


def kernel(points, W_d, b_d, W_a, b_a):
    B, N, _ = points.shape
    assert B == 1 and N == _N
    pts = points[0]
    ptT = pts.T
    wd_p = jnp.concatenate([W_d[:, 0::2], W_d[:, 1::2]], axis=1).T
    wa_p = jnp.concatenate([W_a[:, 0::2], W_a[:, 1::2]], axis=1).T
    bias = (b_d + b_a)[None, :]
    div = np.exp(np.arange(0, _HID, 2, dtype=np.float32)
                 * (-np.log(10000.0) / _HID)).astype(np.float32)[None, :]
    div = jnp.asarray(div)

    out = pl.pallas_call(
        _block_kernel,
        grid=(N // _R,),
        in_specs=[
            pl.BlockSpec((_N, 3), lambda i: (0, 0)),
            pl.BlockSpec((3, _N), lambda i: (0, 0)),
            pl.BlockSpec((_HID, _HID), lambda i: (0, 0)),
            pl.BlockSpec((_HID, _HID), lambda i: (0, 0)),
            pl.BlockSpec((1, _HID), lambda i: (0, 0)),
            pl.BlockSpec((1, _HALF), lambda i: (0, 0)),
        ],
        out_specs=pl.BlockSpec((1, _R, _N, _HID), lambda i: (0, i, 0, 0)),
        out_shape=jax.ShapeDtypeStruct((B, N, N, _HID), jnp.float32),
    )(pts, ptT, wd_p, wa_p, bias, div)
    return out


# speedup vs baseline: 4.3357x; 1.0974x over previous
import numpy as np
import jax
import jax.numpy as jnp
from jax.experimental import pallas as pl
from jax.experimental.pallas import tpu as pltpu

_HID = 256
_HALF = _HID // 2
_SIGMA_D = 0.2
_FACTOR_A = 180.0 / (15.0 * np.pi)
_K = 3
_N = 256
_R = 8


_SINC = (6.278627779e+00, -4.109360634e+01, 7.792988247e+01, -5.608619073e+01)
_COSC = (9.999598318e-01, -1.973104743e+01, 6.467356500e+01, -8.240452437e+01,
         4.564873189e+01)


def _sincos_turns(m):
    f = m - jnp.round(m)
    t = f * f
    s = _SINC[3]
    c = _COSC[4]
    c = c * t + _COSC[3]
    for j in range(2, -1, -1):
        s = s * t + _SINC[j]
        c = c * t + _COSC[j]
    return f * s, c


def _emb_linear(x, w_ref, divm_ref):
    m = x.reshape(_R, _N, 1) * divm_ref[:][None, :, :]
    s, c = _sincos_turns(m)
    s = s.reshape(_R * _N, _HALF)
    c = c.reshape(_R * _N, _HALF)
    return (jnp.dot(s, w_ref[0:_HALF, :], preferred_element_type=jnp.float32)
            + jnp.dot(c, w_ref[_HALF:, :], preferred_element_type=jnp.float32))


def _block_kernel(pts_ref, ptT_ref, wd_ref, wa_ref, bias_ref, divd_ref, diva_ref,
                  out_ref):
    i = pl.program_id(0)
    ptT = ptT_ref[:]
    prow = pts_ref[pl.ds(i * _R, _R), :]

    xy = jnp.dot(prow, ptT, preferred_element_type=jnp.float32)
    x2 = jnp.sum(prow * prow, axis=1, keepdims=True)
    y2 = jnp.sum(ptT * ptT, axis=0, keepdims=True)
    dist = jnp.sqrt(jnp.maximum(x2 - 2.0 * xy + y2, 0.0))

    iota = jax.lax.broadcasted_iota(jnp.int32, (_R, _N), 1)
    px_all = ptT[0:1, :]
    py_all = ptT[1:2, :]
    pz_all = ptT[2:3, :]
    px_row = prow[:, 0:1]
    py_row = prow[:, 1:2]
    pz_row = prow[:, 2:3]
    ax = px_all - px_row
    ay = py_all - py_row
    az = pz_all - pz_row

    d = dist
    refvecs = []
    for t in range(_K + 1):
        vmin = jnp.min(d, axis=1, keepdims=True)
        idx = jnp.min(jnp.where(d == vmin, iota, _N), axis=1, keepdims=True)
        sel = iota == idx
        d = jnp.where(sel, jnp.float32(np.inf), d)
        if t > 0:
            rx = jnp.sum(jnp.where(sel, px_all, 0.0), axis=1, keepdims=True) - px_row
            ry = jnp.sum(jnp.where(sel, py_all, 0.0), axis=1, keepdims=True) - py_row
            rz = jnp.sum(jnp.where(sel, pz_all, 0.0), axis=1, keepdims=True) - pz_row
            refvecs.append((rx, ry, rz))

    out = _emb_linear(dist, wd_ref, divd_ref)

    m = None
    for rx, ry, rz in refvecs:
        cx = ry * az - rz * ay
        cy = rz * ax - rx * az
        cz = rx * ay - ry * ax
        sinv = jnp.sqrt(cx * cx + cy * cy + cz * cz)
        cosv = rx * ax + ry * ay + rz * az
        ang = jnp.arctan2(sinv, cosv)
        e = _emb_linear(ang, wa_ref, diva_ref)
        m = e if m is None else jnp.maximum(m, e)

    out = out + m + bias_ref[:]
    out_ref[0] = out.reshape(_R, _N, _HID)


def kernel(points, W_d, b_d, W_a, b_a):
    B, N, _ = points.shape
    assert B == 1 and N == _N
    pts = points[0]
    ptT = pts.T
    wd_p = jnp.concatenate([W_d[:, 0::2], W_d[:, 1::2]], axis=1).T
    wa_p = jnp.concatenate([W_a[:, 0::2], W_a[:, 1::2]], axis=1).T
    bias = (b_d + b_a)[None, :]
    div = np.exp(np.arange(0, _HID, 2, dtype=np.float32)
                 * (-np.log(10000.0) / _HID)).astype(np.float64)
    inv2pi = 1.0 / (2.0 * np.pi)
    divd = jnp.asarray((div * inv2pi / _SIGMA_D).astype(np.float32)[None, :])
    diva = jnp.asarray((div * inv2pi * _FACTOR_A).astype(np.float32)[None, :])

    out = pl.pallas_call(
        _block_kernel,
        grid=(N // _R,),
        in_specs=[
            pl.BlockSpec((_N, 3), lambda i: (0, 0)),
            pl.BlockSpec((3, _N), lambda i: (0, 0)),
            pl.BlockSpec((_HID, _HID), lambda i: (0, 0)),
            pl.BlockSpec((_HID, _HID), lambda i: (0, 0)),
            pl.BlockSpec((1, _HID), lambda i: (0, 0)),
            pl.BlockSpec((1, _HALF), lambda i: (0, 0)),
            pl.BlockSpec((1, _HALF), lambda i: (0, 0)),
        ],
        out_specs=pl.BlockSpec((1, _R, _N, _HID), lambda i: (0, i, 0, 0)),
        out_shape=jax.ShapeDtypeStruct((B, N, N, _HID), jnp.float32),
    )(pts, ptT, wd_p, wa_p, bias, divd, diva)
    return out


# speedup vs baseline: 4.5836x; 1.0572x over previous
import numpy as np
import jax
import jax.numpy as jnp
from jax.experimental import pallas as pl
from jax.experimental.pallas import tpu as pltpu

_HID = 256
_HALF = _HID // 2
_SIGMA_D = 0.2
_FACTOR_A = 180.0 / (15.0 * np.pi)
_K = 3
_N = 256
_R = 16


_SINC = (6.278627779e+00, -4.109360634e+01, 7.792988247e+01, -5.608619073e+01)
_COSC = (9.999598318e-01, -1.973104743e+01, 6.467356500e+01, -8.240452437e+01,
         4.564873189e+01)


def _sincos_turns(m):
    f = m - jnp.round(m)
    t = f * f
    s = _SINC[3]
    c = _COSC[4]
    c = c * t + _COSC[3]
    for j in range(2, -1, -1):
        s = s * t + _SINC[j]
        c = c * t + _COSC[j]
    return f * s, c


def _emb_linear(x, w_ref, divm_ref):
    m = x.reshape(_R, _N, 1) * divm_ref[:][None, :, :]
    s, c = _sincos_turns(m)
    s = s.reshape(_R * _N, _HALF)
    c = c.reshape(_R * _N, _HALF)
    return (jnp.dot(s, w_ref[0:_HALF, :], preferred_element_type=jnp.float32)
            + jnp.dot(c, w_ref[_HALF:, :], preferred_element_type=jnp.float32))


def _block_kernel(pts_ref, ptT_ref, wd_ref, wa_ref, bias_ref, divd_ref, diva_ref,
                  out_ref):
    i = pl.program_id(0)
    ptT = ptT_ref[:]
    prow = pts_ref[pl.ds(i * _R, _R), :]

    xy = jnp.dot(prow, ptT, preferred_element_type=jnp.float32)
    x2 = jnp.sum(prow * prow, axis=1, keepdims=True)
    y2 = jnp.sum(ptT * ptT, axis=0, keepdims=True)
    dist = jnp.sqrt(jnp.maximum(x2 - 2.0 * xy + y2, 0.0))

    iota = jax.lax.broadcasted_iota(jnp.int32, (_R, _N), 1)
    px_all = ptT[0:1, :]
    py_all = ptT[1:2, :]
    pz_all = ptT[2:3, :]
    px_row = prow[:, 0:1]
    py_row = prow[:, 1:2]
    pz_row = prow[:, 2:3]
    ax = px_all - px_row
    ay = py_all - py_row
    az = pz_all - pz_row

    d = dist
    refvecs = []
    for t in range(_K + 1):
        vmin = jnp.min(d, axis=1, keepdims=True)
        idx = jnp.min(jnp.where(d == vmin, iota, _N), axis=1, keepdims=True)
        sel = iota == idx
        d = jnp.where(sel, jnp.float32(np.inf), d)
        if t > 0:
            rx = jnp.sum(jnp.where(sel, px_all, 0.0), axis=1, keepdims=True) - px_row
            ry = jnp.sum(jnp.where(sel, py_all, 0.0), axis=1, keepdims=True) - py_row
            rz = jnp.sum(jnp.where(sel, pz_all, 0.0), axis=1, keepdims=True) - pz_row
            refvecs.append((rx, ry, rz))

    out = _emb_linear(dist, wd_ref, divd_ref)

    m = None
    for rx, ry, rz in refvecs:
        cx = ry * az - rz * ay
        cy = rz * ax - rx * az
        cz = rx * ay - ry * ax
        sinv = jnp.sqrt(cx * cx + cy * cy + cz * cz)
        cosv = rx * ax + ry * ay + rz * az
        ang = jnp.arctan2(sinv, cosv)
        e = _emb_linear(ang, wa_ref, diva_ref)
        m = e if m is None else jnp.maximum(m, e)

    out = out + m + bias_ref[:]
    out_ref[0] = out.reshape(_R, _N, _HID)


def kernel(points, W_d, b_d, W_a, b_a):
    B, N, _ = points.shape
    assert B == 1 and N == _N
    pts = points[0]
    ptT = pts.T
    wd_p = jnp.concatenate([W_d[:, 0::2], W_d[:, 1::2]], axis=1).T
    wa_p = jnp.concatenate([W_a[:, 0::2], W_a[:, 1::2]], axis=1).T
    bias = (b_d + b_a)[None, :]
    div = np.exp(np.arange(0, _HID, 2, dtype=np.float32)
                 * (-np.log(10000.0) / _HID)).astype(np.float64)
    inv2pi = 1.0 / (2.0 * np.pi)
    divd = jnp.asarray((div * inv2pi / _SIGMA_D).astype(np.float32)[None, :])
    diva = jnp.asarray((div * inv2pi * _FACTOR_A).astype(np.float32)[None, :])

    out = pl.pallas_call(
        _block_kernel,
        grid=(N // _R,),
        in_specs=[
            pl.BlockSpec((_N, 3), lambda i: (0, 0)),
            pl.BlockSpec((3, _N), lambda i: (0, 0)),
            pl.BlockSpec((_HID, _HID), lambda i: (0, 0)),
            pl.BlockSpec((_HID, _HID), lambda i: (0, 0)),
            pl.BlockSpec((1, _HID), lambda i: (0, 0)),
            pl.BlockSpec((1, _HALF), lambda i: (0, 0)),
            pl.BlockSpec((1, _HALF), lambda i: (0, 0)),
        ],
        out_specs=pl.BlockSpec((1, _R, _N, _HID), lambda i: (0, i, 0, 0)),
        out_shape=jax.ShapeDtypeStruct((B, N, N, _HID), jnp.float32),
    )(pts, ptT, wd_p, wa_p, bias, divd, diva)
    return out


# speedup vs baseline: 4.7036x; 1.0262x over previous
import numpy as np
import jax
import jax.numpy as jnp
from jax.experimental import pallas as pl
from jax.experimental.pallas import tpu as pltpu

_HID = 256
_HALF = _HID // 2
_SIGMA_D = 0.2
_FACTOR_A = 180.0 / (15.0 * np.pi)
_K = 3
_N = 256
_R = 32


_SINC = (6.278627779e+00, -4.109360634e+01, 7.792988247e+01, -5.608619073e+01)
_COSC = (9.999598318e-01, -1.973104743e+01, 6.467356500e+01, -8.240452437e+01,
         4.564873189e+01)


def _sincos_turns(m):
    f = m - jnp.round(m)
    t = f * f
    s = _SINC[3]
    c = _COSC[4]
    c = c * t + _COSC[3]
    for j in range(2, -1, -1):
        s = s * t + _SINC[j]
        c = c * t + _COSC[j]
    return f * s, c


def _emb_linear(x, w_ref, divm_ref):
    m = x.reshape(_R, _N, 1) * divm_ref[:][None, :, :]
    s, c = _sincos_turns(m)
    s = s.reshape(_R * _N, _HALF)
    c = c.reshape(_R * _N, _HALF)
    return (jnp.dot(s, w_ref[0:_HALF, :], preferred_element_type=jnp.float32)
            + jnp.dot(c, w_ref[_HALF:, :], preferred_element_type=jnp.float32))


def _block_kernel(pts_ref, ptT_ref, wd_ref, wa_ref, bias_ref, divd_ref, diva_ref,
                  out_ref):
    i = pl.program_id(0)
    ptT = ptT_ref[:]
    prow = pts_ref[pl.ds(i * _R, _R), :]

    xy = jnp.dot(prow, ptT, preferred_element_type=jnp.float32)
    x2 = jnp.sum(prow * prow, axis=1, keepdims=True)
    y2 = jnp.sum(ptT * ptT, axis=0, keepdims=True)
    dist = jnp.sqrt(jnp.maximum(x2 - 2.0 * xy + y2, 0.0))

    iota = jax.lax.broadcasted_iota(jnp.int32, (_R, _N), 1)
    px_all = ptT[0:1, :]
    py_all = ptT[1:2, :]
    pz_all = ptT[2:3, :]
    px_row = prow[:, 0:1]
    py_row = prow[:, 1:2]
    pz_row = prow[:, 2:3]
    ax = px_all - px_row
    ay = py_all - py_row
    az = pz_all - pz_row

    d = dist
    refvecs = []
    for t in range(_K + 1):
        vmin = jnp.min(d, axis=1, keepdims=True)
        idx = jnp.min(jnp.where(d == vmin, iota, _N), axis=1, keepdims=True)
        sel = iota == idx
        d = jnp.where(sel, jnp.float32(np.inf), d)
        if t > 0:
            rx = jnp.sum(jnp.where(sel, px_all, 0.0), axis=1, keepdims=True) - px_row
            ry = jnp.sum(jnp.where(sel, py_all, 0.0), axis=1, keepdims=True) - py_row
            rz = jnp.sum(jnp.where(sel, pz_all, 0.0), axis=1, keepdims=True) - pz_row
            refvecs.append((rx, ry, rz))

    out = _emb_linear(dist, wd_ref, divd_ref)

    m = None
    for rx, ry, rz in refvecs:
        cx = ry * az - rz * ay
        cy = rz * ax - rx * az
        cz = rx * ay - ry * ax
        sinv = jnp.sqrt(cx * cx + cy * cy + cz * cz)
        cosv = rx * ax + ry * ay + rz * az
        ang = jnp.arctan2(sinv, cosv)
        e = _emb_linear(ang, wa_ref, diva_ref)
        m = e if m is None else jnp.maximum(m, e)

    out = out + m + bias_ref[:]
    out_ref[0] = out.reshape(_R, _N, _HID)


def kernel(points, W_d, b_d, W_a, b_a):
    B, N, _ = points.shape
    assert B == 1 and N == _N
    pts = points[0]
    ptT = pts.T
    wd_p = jnp.concatenate([W_d[:, 0::2], W_d[:, 1::2]], axis=1).T
    wa_p = jnp.concatenate([W_a[:, 0::2], W_a[:, 1::2]], axis=1).T
    bias = (b_d + b_a)[None, :]
    div = np.exp(np.arange(0, _HID, 2, dtype=np.float32)
                 * (-np.log(10000.0) / _HID)).astype(np.float64)
    inv2pi = 1.0 / (2.0 * np.pi)
    divd = jnp.asarray((div * inv2pi / _SIGMA_D).astype(np.float32)[None, :])
    diva = jnp.asarray((div * inv2pi * _FACTOR_A).astype(np.float32)[None, :])

    out = pl.pallas_call(
        _block_kernel,
        grid=(N // _R,),
        in_specs=[
            pl.BlockSpec((_N, 3), lambda i: (0, 0)),
            pl.BlockSpec((3, _N), lambda i: (0, 0)),
            pl.BlockSpec((_HID, _HID), lambda i: (0, 0)),
            pl.BlockSpec((_HID, _HID), lambda i: (0, 0)),
            pl.BlockSpec((1, _HID), lambda i: (0, 0)),
            pl.BlockSpec((1, _HALF), lambda i: (0, 0)),
            pl.BlockSpec((1, _HALF), lambda i: (0, 0)),
        ],
        out_specs=pl.BlockSpec((1, _R, _N, _HID), lambda i: (0, i, 0, 0)),
        out_shape=jax.ShapeDtypeStruct((B, N, N, _HID), jnp.float32),
    )(pts, ptT, wd_p, wa_p, bias, divd, diva)
    return out
